# baseline (speedup 1.0000x reference)
import functools

import jax
import jax.numpy as jnp
from jax import lax
from jax.experimental import pallas as pl
from jax.experimental.pallas import tpu as pltpu
from jax.experimental.pallas import tpu_sc as plsc

N = 10000
E = 320000
F = 128
H = 128
C = 10
G = 16

NC = 2
NS = 16
NW = NC * NS
EPW = E // NW
K = 40
NCHUNK = EPW // K
LOOP_END = ((NCHUNK - 2) // 4) * 4
OS = 624
LAST_OS = N - 15 * OS

_f32 = jnp.float32


def _dot(a, b):
    return jnp.dot(a, b, preferred_element_type=_f32)


def _tc_fold1(x_ref, W2_ref, b2_ref, We_ref, be_ref, p_ref, q_ref):
    x = x_ref[...]
    We = We_ref[...]
    A = _dot(W2_ref[0:F, :], We)
    B = _dot(W2_ref[F:2 * F, :], We)
    c = _dot(b2_ref[...], We) + be_ref[...]
    p_ref[...] = x + _dot(x, A)
    q_ref[...] = _dot(x, B) + c


def _bn_mlp(u, Wa, ba, Wb, bb):
    t = _dot(u, Wa) + ba
    mu = jnp.mean(t, axis=0, keepdims=True)
    tc = t - mu
    var = jnp.mean(tc * tc, axis=0, keepdims=True)
    r = jnp.maximum(tc / jnp.sqrt(var + 1e-5), 0.0)
    return _dot(r, Wb) + bb


def _tc_mid(x_ref, pa_ref, W1a_ref, b1a_ref, W1b_ref, b1b_ref,
            W3_ref, b3_ref, We2_ref, be2_ref, p_ref, q_ref, xr_ref):
    u = x_ref[...] + pa_ref[0] + pa_ref[1]
    h = _bn_mlp(u, W1a_ref[...], b1a_ref[...], W1b_ref[...], b1b_ref[...])
    xr = jnp.maximum(h, 0.0)
    We2 = We2_ref[...]
    A = _dot(W3_ref[0:H, :], We2)
    B = _dot(W3_ref[H:2 * H, :], We2)
    c = _dot(b3_ref[...], We2) + be2_ref[...]
    p_ref[...] = xr + _dot(h, A)
    q_ref[...] = _dot(h, B) + c
    xr_ref[...] = xr


def _tc_final(xr_ref, pb_ref, W2a_ref, b2a_ref, W2b_ref, b2b_ref,
              batch_ref, Wl1_ref, bl1_ref, out_ref):
    u = xr_ref[...] + pb_ref[0] + pb_ref[1]
    h2 = _bn_mlp(u, W2a_ref[...], b2a_ref[...], W2b_ref[...], b2b_ref[...])
    hr = jnp.maximum(h2, 0.0)
    onehot = (batch_ref[...] ==
              lax.broadcasted_iota(jnp.int32, (N, G), 1)).astype(_f32)
    pooled = lax.dot_general(onehot, hr, (((0,), (0,)), ((), ())),
                             preferred_element_type=_f32)
    logits = _dot(pooled, Wl1_ref[...]) + bl1_ref[...]
    m = jnp.max(logits, axis=1, keepdims=True)
    lse = jnp.log(jnp.sum(jnp.exp(logits - m), axis=1, keepdims=True)) + m
    out_ref[...] = logits - lse


def _sc_edge_body(p_hbm, q_hbm, eidx_hbm, zero_hbm, out_hbm,
                  aggr_sh, idx4,
                  pb0, qb0, mb0, pb1, qb1, mb1,
                  sp0, sq0, ss0, sp1, sq1, ss1,
                  si0, si1, si2, si3):
    c = lax.axis_index("c")
    s = lax.axis_index("s")
    w = c * NS + s
    bufs = ((pb0, qb0, mb0, sp0, sq0, ss0), (pb1, qb1, mb1, sp1, sq1, ss1))
    sis = (si0, si1, si2, si3)

    def _idx_issue(kk, slot):
        pltpu.async_copy(eidx_hbm.at[w, pl.ds(2 * kk, 2)], idx4.at[slot],
                         sis[slot])

    def _idx_wait(kk, slot):
        pltpu.make_async_copy(eidx_hbm.at[w, pl.ds(2 * kk, 2)], idx4.at[slot],
                              sis[slot]).wait()

    def _gather_issue(slot, b):
        pb, qb, _, sp, sq, _ = bufs[b]
        pltpu.async_copy(p_hbm.at[idx4.at[slot, 0]], pb, sp)
        pltpu.async_copy(q_hbm.at[idx4.at[slot, 1]], qb, sq)

    def _gather_wait(slot, b):
        pb, qb, _, sp, sq, _ = bufs[b]
        pltpu.make_async_copy(p_hbm.at[idx4.at[slot, 0]], pb, sp).wait()
        pltpu.make_async_copy(q_hbm.at[idx4.at[slot, 1]], qb, sq).wait()

    def _scatter_drain(slot, b):
        mb, ss = bufs[b][2], bufs[b][5]
        pltpu.make_async_copy(mb, aggr_sh.at[idx4.at[slot, 1]], ss).wait()

    def _relu_sum(b):
        pb, qb, mb = bufs[b][0], bufs[b][1], bufs[b][2]

        @plsc.parallel_loop(0, K, 1, unroll=4)
        def _row(r):
            for j in range(8):
                sl = pl.ds(j * 16, 16)
                mb[r, sl] = jnp.maximum(pb[r, sl] + qb[r, sl], 0.0)

    _idx_issue(0, 0)
    _idx_issue(1, 1)
    off = pl.multiple_of(s * OS, 8)

    @pl.when(s < NS - 1)
    def _zero():
        pltpu.sync_copy(zero_hbm.at[pl.ds(0, OS)], aggr_sh.at[pl.ds(off, OS)])

    @pl.when(s == NS - 1)
    def _zero_last():
        pltpu.sync_copy(zero_hbm.at[pl.ds(0, LAST_OS)],
                        aggr_sh.at[pl.ds(off, LAST_OS)])

    plsc.subcore_barrier()

    _idx_wait(0, 0)
    _gather_issue(0, 0)
    _idx_issue(2, 2)

    @pl.loop(0, LOOP_END, step=4)
    def _main(k):
        for b in range(4):
            b2 = b % 2
            kk = k + b
            _gather_wait(b, b2)
            if b < 2:
                @pl.when(k > 0)
                def _drain():
                    _scatter_drain((b + 2) % 4, b2)
            else:
                _scatter_drain((b + 2) % 4, b2)
            _relu_sum(b2)
            mb, ss = bufs[b2][2], bufs[b2][5]
            pltpu.async_copy(mb, aggr_sh.at[idx4.at[b, 1]], ss, add=True)
            _idx_wait(kk + 1, (b + 1) % 4)
            _gather_issue((b + 1) % 4, 1 - b2)
            _idx_issue(kk + 2, (b + 2) % 4)

    for kk in range(LOOP_END, NCHUNK):
        slot = kk % 4
        b2 = kk % 2
        _gather_wait(slot, b2)
        _scatter_drain((slot + 2) % 4, b2)
        _relu_sum(b2)
        mb, ss = bufs[b2][2], bufs[b2][5]
        pltpu.async_copy(mb, aggr_sh.at[idx4.at[slot, 1]], ss, add=True)
        if kk + 1 < NCHUNK:
            _idx_wait(kk + 1, (slot + 1) % 4)
            _gather_issue((slot + 1) % 4, 1 - b2)
        if kk + 2 < NCHUNK:
            _idx_issue(kk + 2, (slot + 2) % 4)
    for kk in (NCHUNK - 2, NCHUNK - 1):
        _scatter_drain(kk % 4, kk % 2)

    plsc.subcore_barrier()
    off = pl.multiple_of(s * OS, 8)

    @pl.when(s < NS - 1)
    def _copy_out():
        pltpu.sync_copy(aggr_sh.at[pl.ds(off, OS)],
                        out_hbm.at[c, pl.ds(off, OS)])

    @pl.when(s == NS - 1)
    def _copy_out_last():
        pltpu.sync_copy(aggr_sh.at[pl.ds(off, LAST_OS)],
                        out_hbm.at[c, pl.ds(off, LAST_OS)])


@functools.cache
def _get_sc_edge():
    return pl.kernel(
        _sc_edge_body,
        out_type=jax.ShapeDtypeStruct((NC, N, F), _f32),
        mesh=plsc.VectorSubcoreMesh(core_axis_name="c", subcore_axis_name="s",
                                    num_cores=NC, num_subcores=NS),
        scratch_types=(
            [pltpu.VMEM_SHARED((N, F), _f32)]
            + [pltpu.VMEM((4, 2, K), jnp.int32)]
            + [pltpu.VMEM((K, F), _f32)] * 6
            + [pltpu.SemaphoreType.DMA] * 10
        ),
    )


def kernel(x, edge_index, batch, W_lin2, b_lin2, We1, be1, W1a, b1a, W1b, b1b,
           W_lin3, b_lin3, We2, be2, W2a, b2a, W2b, b2b, W_lin1, b_lin1):
    eidx = (edge_index.reshape(2, NW, NCHUNK, K)
            .transpose(1, 2, 0, 3).reshape(NW, 2 * NCHUNK, K))
    zero = jnp.zeros((LAST_OS, F), _f32)
    sds = jax.ShapeDtypeStruct

    p1, q1 = pl.pallas_call(
        _tc_fold1,
        out_shape=[sds((N, F), _f32), sds((N, F), _f32)],
    )(x, W_lin2, b_lin2.reshape(1, H), We1, be1.reshape(1, F))

    sc_edge = _get_sc_edge()
    pa = sc_edge(p1, q1, eidx, zero)

    p2, q2, xr = pl.pallas_call(
        _tc_mid,
        out_shape=[sds((N, H), _f32), sds((N, H), _f32), sds((N, H), _f32)],
    )(x, pa, W1a, b1a.reshape(1, H), W1b, b1b.reshape(1, H),
      W_lin3, b_lin3.reshape(1, H), We2, be2.reshape(1, H))

    pb = sc_edge(p2, q2, eidx, zero)

    out = pl.pallas_call(
        _tc_final,
        out_shape=sds((G, C), _f32),
    )(xr, pb, W2a, b2a.reshape(1, H), W2b, b2b.reshape(1, H),
      batch.reshape(N, 1), W_lin1, b_lin1.reshape(1, C))

    return out


# speedup vs baseline: 1.3135x; 1.3135x over previous
import functools

import jax
import jax.numpy as jnp
import numpy as np
from jax import lax
from jax.experimental import pallas as pl
from jax.experimental.pallas import tpu as pltpu
from jax.experimental.pallas import tpu_sc as plsc

N = 10000
E = 320000
F = 128
H = 128
C = 10
G = 16

NC = 2
NS = 16
NW = NC * NS
EPW = E // NW
K = 80
NCHUNK = EPW // K
LOOP_END = ((NCHUNK - 2) // 4) * 4
OS = 624
LAST_OS = N - 15 * OS

_f32 = jnp.float32
_bf16 = jnp.bfloat16

_PERM = np.concatenate(
    [32 * g + np.concatenate([np.arange(0, 32, 2), np.arange(1, 32, 2)])
     for g in range(F // 32)])


def _dot(a, b):
    return jnp.dot(a, b, preferred_element_type=_f32)


def _tc_fold1(x_ref, W2_ref, b2_ref, We_ref, be_ref, p_ref, q_ref):
    x = x_ref[...]
    We = We_ref[...]
    A = _dot(W2_ref[0:F, :], We)
    B = _dot(W2_ref[F:2 * F, :], We)
    c = _dot(b2_ref[...], We) + be_ref[...]
    p_ref[...] = (x + _dot(x, A)).astype(_bf16)
    q_ref[...] = (_dot(x, B) + c).astype(_bf16)


def _bn_mlp(t, Wb, bb):
    mu = jnp.mean(t, axis=0, keepdims=True)
    tc = t - mu
    var = jnp.mean(tc * tc, axis=0, keepdims=True)
    r = jnp.maximum(tc / jnp.sqrt(var + 1e-5), 0.0)
    return _dot(r, Wb) + bb


def _tc_mid(x_ref, pa_ref, W1a_ref, W1ap_ref, b1a_ref, W1b_ref, b1b_ref,
            W3_ref, b3_ref, We2_ref, be2_ref, p_ref, q_ref, xr_ref):
    ag = pa_ref[0] + pa_ref[1]
    t = _dot(x_ref[...], W1a_ref[...]) + _dot(ag, W1ap_ref[...]) + b1a_ref[...]
    h = _bn_mlp(t, W1b_ref[...], b1b_ref[...])
    xr = jnp.maximum(h, 0.0)
    We2 = We2_ref[...]
    A = _dot(W3_ref[0:H, :], We2)
    B = _dot(W3_ref[H:2 * H, :], We2)
    c = _dot(b3_ref[...], We2) + be2_ref[...]
    p_ref[...] = (xr + _dot(h, A)).astype(_bf16)
    q_ref[...] = (_dot(h, B) + c).astype(_bf16)
    xr_ref[...] = xr


def _tc_final(xr_ref, pb_ref, W2a_ref, W2ap_ref, b2a_ref, W2b_ref, b2b_ref,
              batch_ref, Wl1_ref, bl1_ref, out_ref):
    ag = pb_ref[0] + pb_ref[1]
    t = (_dot(xr_ref[...], W2a_ref[...]) + _dot(ag, W2ap_ref[...])
         + b2a_ref[...])
    h2 = _bn_mlp(t, W2b_ref[...], b2b_ref[...])
    hr = jnp.maximum(h2, 0.0)
    onehot = (batch_ref[...] ==
              lax.broadcasted_iota(jnp.int32, (N, G), 1)).astype(_f32)
    pooled = lax.dot_general(onehot, hr, (((0,), (0,)), ((), ())),
                             preferred_element_type=_f32)
    logits = _dot(pooled, Wl1_ref[...]) + bl1_ref[...]
    m = jnp.max(logits, axis=1, keepdims=True)
    lse = jnp.log(jnp.sum(jnp.exp(logits - m), axis=1, keepdims=True)) + m
    out_ref[...] = logits - lse


def _sc_edge_body(p_hbm, q_hbm, eidx_hbm, zero_hbm, out_hbm,
                  aggr_sh, idx4,
                  pb0, qb0, mb0, pb1, qb1, mb1,
                  sp0, sq0, ss0, sp1, sq1, ss1,
                  si0, si1, si2, si3):
    c = lax.axis_index("c")
    s = lax.axis_index("s")
    w = c * NS + s
    bufs = ((pb0, qb0, mb0, sp0, sq0, ss0), (pb1, qb1, mb1, sp1, sq1, ss1))
    sis = (si0, si1, si2, si3)

    def _idx_issue(kk, slot):
        pltpu.async_copy(eidx_hbm.at[w, pl.ds(2 * kk, 2)], idx4.at[slot],
                         sis[slot])

    def _idx_wait(kk, slot):
        pltpu.make_async_copy(eidx_hbm.at[w, pl.ds(2 * kk, 2)], idx4.at[slot],
                              sis[slot]).wait()

    def _gather_issue(slot, b):
        pb, qb, _, sp, sq, _ = bufs[b]
        pltpu.async_copy(p_hbm.at[idx4.at[slot, 0]], pb, sp)
        pltpu.async_copy(q_hbm.at[idx4.at[slot, 1]], qb, sq)

    def _gather_wait(slot, b):
        pb, qb, _, sp, sq, _ = bufs[b]
        pltpu.make_async_copy(p_hbm.at[idx4.at[slot, 0]], pb, sp).wait()
        pltpu.make_async_copy(q_hbm.at[idx4.at[slot, 1]], qb, sq).wait()

    def _scatter_drain(slot, b):
        mb, ss = bufs[b][2], bufs[b][5]
        pltpu.make_async_copy(mb, aggr_sh.at[idx4.at[slot, 1]], ss).wait()

    def _relu_sum(b):
        pb, qb, mb = bufs[b][0], bufs[b][1], bufs[b][2]

        himask = jnp.int32(-65536)

        @plsc.parallel_loop(0, K, 1, unroll=4)
        def _row(r):
            for g in range(F // 32):
                pw = pb[r, pl.ds(g * 16, 16)]
                qw = qb[r, pl.ds(g * 16, 16)]
                plo = lax.bitcast_convert_type(pw << 16, _f32)
                qlo = lax.bitcast_convert_type(qw << 16, _f32)
                phi = lax.bitcast_convert_type(pw & himask, _f32)
                qhi = lax.bitcast_convert_type(qw & himask, _f32)
                mb[r, pl.ds(g * 32, 16)] = jnp.maximum(plo + qlo, 0.0)
                mb[r, pl.ds(g * 32 + 16, 16)] = jnp.maximum(phi + qhi, 0.0)

    _idx_issue(0, 0)
    _idx_issue(1, 1)
    off = pl.multiple_of(s * OS, 8)

    @pl.when(s < NS - 1)
    def _zero():
        pltpu.sync_copy(zero_hbm.at[pl.ds(0, OS)], aggr_sh.at[pl.ds(off, OS)])

    @pl.when(s == NS - 1)
    def _zero_last():
        pltpu.sync_copy(zero_hbm.at[pl.ds(0, LAST_OS)],
                        aggr_sh.at[pl.ds(off, LAST_OS)])

    plsc.subcore_barrier()

    _idx_wait(0, 0)
    _gather_issue(0, 0)
    _idx_issue(2, 2)

    @pl.loop(0, LOOP_END, step=4)
    def _main(k):
        for b in range(4):
            b2 = b % 2
            kk = k + b
            _gather_wait(b, b2)
            if b < 2:
                @pl.when(k > 0)
                def _drain():
                    _scatter_drain((b + 2) % 4, b2)
            else:
                _scatter_drain((b + 2) % 4, b2)
            _relu_sum(b2)
            mb, ss = bufs[b2][2], bufs[b2][5]
            pltpu.async_copy(mb, aggr_sh.at[idx4.at[b, 1]], ss, add=True)
            _idx_wait(kk + 1, (b + 1) % 4)
            _gather_issue((b + 1) % 4, 1 - b2)
            _idx_issue(kk + 2, (b + 2) % 4)

    for kk in range(LOOP_END, NCHUNK):
        slot = kk % 4
        b2 = kk % 2
        _gather_wait(slot, b2)
        _scatter_drain((slot + 2) % 4, b2)
        _relu_sum(b2)
        mb, ss = bufs[b2][2], bufs[b2][5]
        pltpu.async_copy(mb, aggr_sh.at[idx4.at[slot, 1]], ss, add=True)
        if kk + 1 < NCHUNK:
            _idx_wait(kk + 1, (slot + 1) % 4)
            _gather_issue((slot + 1) % 4, 1 - b2)
        if kk + 2 < NCHUNK:
            _idx_issue(kk + 2, (slot + 2) % 4)
    for kk in (NCHUNK - 2, NCHUNK - 1):
        _scatter_drain(kk % 4, kk % 2)

    plsc.subcore_barrier()
    off = pl.multiple_of(s * OS, 8)

    @pl.when(s < NS - 1)
    def _copy_out():
        pltpu.sync_copy(aggr_sh.at[pl.ds(off, OS)],
                        out_hbm.at[c, pl.ds(off, OS)])

    @pl.when(s == NS - 1)
    def _copy_out_last():
        pltpu.sync_copy(aggr_sh.at[pl.ds(off, LAST_OS)],
                        out_hbm.at[c, pl.ds(off, LAST_OS)])


@functools.cache
def _get_sc_edge():
    return pl.kernel(
        _sc_edge_body,
        out_type=jax.ShapeDtypeStruct((NC, N, F), _f32),
        mesh=plsc.VectorSubcoreMesh(core_axis_name="c", subcore_axis_name="s",
                                    num_cores=NC, num_subcores=NS),
        compiler_params=pltpu.CompilerParams(use_tc_tiling_on_sc=False),
        scratch_types=(
            [pltpu.VMEM_SHARED((N, F), _f32)]
            + [pltpu.VMEM((4, 2, K), jnp.int32)]
            + [pltpu.VMEM((K, F // 2), jnp.int32),
               pltpu.VMEM((K, F // 2), jnp.int32),
               pltpu.VMEM((K, F), _f32)] * 2
            + [pltpu.SemaphoreType.DMA] * 10
        ),
    )


def kernel(x, edge_index, batch, W_lin2, b_lin2, We1, be1, W1a, b1a, W1b, b1b,
           W_lin3, b_lin3, We2, be2, W2a, b2a, W2b, b2b, W_lin1, b_lin1):
    eidx = (edge_index.reshape(2, NW, NCHUNK, K)
            .transpose(1, 2, 0, 3).reshape(NW, 2 * NCHUNK, K))
    zero = jnp.zeros((LAST_OS, F), _f32)
    sds = jax.ShapeDtypeStruct

    def _pack(t):
        return lax.bitcast_convert_type(t.reshape(N, F // 2, 2), jnp.int32)

    p1, q1 = pl.pallas_call(
        _tc_fold1,
        out_shape=[sds((N, F), _bf16), sds((N, F), _bf16)],
    )(x, W_lin2, b_lin2.reshape(1, H), We1, be1.reshape(1, F))

    sc_edge = _get_sc_edge()
    pa = sc_edge(_pack(p1), _pack(q1), eidx, zero)

    p2, q2, xr = pl.pallas_call(
        _tc_mid,
        out_shape=[sds((N, H), _bf16), sds((N, H), _bf16), sds((N, H), _f32)],
    )(x, pa, W1a, W1a[_PERM], b1a.reshape(1, H), W1b, b1b.reshape(1, H),
      W_lin3, b_lin3.reshape(1, H), We2, be2.reshape(1, H))

    pb = sc_edge(_pack(p2), _pack(q2), eidx, zero)

    out = pl.pallas_call(
        _tc_final,
        out_shape=sds((G, C), _f32),
    )(xr, pb, W2a, W2a[_PERM], b2a.reshape(1, H), W2b, b2b.reshape(1, H),
      batch.reshape(N, 1), W_lin1, b_lin1.reshape(1, C))

    return out


# speedup vs baseline: 1.6018x; 1.2195x over previous
import functools

import jax
import jax.numpy as jnp
import numpy as np
from jax import lax
from jax.experimental import pallas as pl
from jax.experimental.pallas import tpu as pltpu
from jax.experimental.pallas import tpu_sc as plsc

N = 10000
E = 320000
F = 128
H = 128
C = 10
G = 16

NC = 2
NS = 16
NW = NC * NS
EPW = E // NW
K = 80
NCHUNK = EPW // K
LOOP_END = ((NCHUNK - 2) // 4) * 4
OS = 624
LAST_OS = N - 15 * OS

_f32 = jnp.float32
_bf16 = jnp.bfloat16

_PERM = np.concatenate(
    [np.concatenate([16 * g + np.arange(16), 64 + 16 * g + np.arange(16)])
     for g in range(F // 32)])


def _dot(a, b):
    return jnp.dot(a, b, preferred_element_type=_f32)


def _pack_tc(t):
    b = lax.bitcast_convert_type(t, jnp.int32)
    r = (b + 0x7FFF + ((b >> 16) & 1)) >> 16
    return (r[:, :F // 2] & 0xFFFF) | (r[:, F // 2:] << 16)


def _tc_fold1(x_ref, W2_ref, b2_ref, We_ref, be_ref, p_ref, q_ref):
    x = x_ref[...]
    We = We_ref[...]
    A = _dot(W2_ref[0:F, :], We)
    B = _dot(W2_ref[F:2 * F, :], We)
    c = _dot(b2_ref[...], We) + be_ref[...]
    p_ref[...] = _pack_tc(x + _dot(x, A))
    q_ref[...] = _pack_tc(_dot(x, B) + c)


def _bn_mlp(t, Wb, bb):
    mu = jnp.mean(t, axis=0, keepdims=True)
    tc = t - mu
    var = jnp.mean(tc * tc, axis=0, keepdims=True)
    r = jnp.maximum(tc / jnp.sqrt(var + 1e-5), 0.0)
    return _dot(r, Wb) + bb


def _tc_mid(x_ref, pa_ref, W1a_ref, W1ap_ref, b1a_ref, W1b_ref, b1b_ref,
            W3_ref, b3_ref, We2_ref, be2_ref, p_ref, q_ref, xr_ref):
    ag = pa_ref[0] + pa_ref[1]
    t = _dot(x_ref[...], W1a_ref[...]) + _dot(ag, W1ap_ref[...]) + b1a_ref[...]
    h = _bn_mlp(t, W1b_ref[...], b1b_ref[...])
    xr = jnp.maximum(h, 0.0)
    We2 = We2_ref[...]
    A = _dot(W3_ref[0:H, :], We2)
    B = _dot(W3_ref[H:2 * H, :], We2)
    c = _dot(b3_ref[...], We2) + be2_ref[...]
    p_ref[...] = _pack_tc(xr + _dot(h, A))
    q_ref[...] = _pack_tc(_dot(h, B) + c)
    xr_ref[...] = xr


def _tc_final(xr_ref, pb_ref, W2a_ref, W2ap_ref, b2a_ref, W2b_ref, b2b_ref,
              batch_ref, Wl1_ref, bl1_ref, out_ref):
    ag = pb_ref[0] + pb_ref[1]
    t = (_dot(xr_ref[...], W2a_ref[...]) + _dot(ag, W2ap_ref[...])
         + b2a_ref[...])
    h2 = _bn_mlp(t, W2b_ref[...], b2b_ref[...])
    hr = jnp.maximum(h2, 0.0)
    onehot = (batch_ref[...] ==
              lax.broadcasted_iota(jnp.int32, (N, G), 1)).astype(_f32)
    pooled = lax.dot_general(onehot, hr, (((0,), (0,)), ((), ())),
                             preferred_element_type=_f32)
    logits = _dot(pooled, Wl1_ref[...]) + bl1_ref[...]
    m = jnp.max(logits, axis=1, keepdims=True)
    lse = jnp.log(jnp.sum(jnp.exp(logits - m), axis=1, keepdims=True)) + m
    out_ref[...] = logits - lse


def _sc_edge_body(p_hbm, q_hbm, eidx_hbm, zero_hbm, out_hbm,
                  aggr_sh, idx4,
                  pb0, qb0, mb0, pb1, qb1, mb1,
                  sp0, sq0, ss0, sp1, sq1, ss1,
                  si0, si1, si2, si3):
    c = lax.axis_index("c")
    s = lax.axis_index("s")
    w = c * NS + s
    bufs = ((pb0, qb0, mb0, sp0, sq0, ss0), (pb1, qb1, mb1, sp1, sq1, ss1))
    sis = (si0, si1, si2, si3)

    def _idx_issue(kk, slot):
        pltpu.async_copy(eidx_hbm.at[w, pl.ds(2 * kk, 2)], idx4.at[slot],
                         sis[slot])

    def _idx_wait(kk, slot):
        pltpu.make_async_copy(eidx_hbm.at[w, pl.ds(2 * kk, 2)], idx4.at[slot],
                              sis[slot]).wait()

    def _gather_issue(slot, b):
        pb, qb, _, sp, sq, _ = bufs[b]
        pltpu.async_copy(p_hbm.at[idx4.at[slot, 0]], pb, sp)
        pltpu.async_copy(q_hbm.at[idx4.at[slot, 1]], qb, sq)

    def _gather_wait(slot, b):
        pb, qb, _, sp, sq, _ = bufs[b]
        pltpu.make_async_copy(p_hbm.at[idx4.at[slot, 0]], pb, sp).wait()
        pltpu.make_async_copy(q_hbm.at[idx4.at[slot, 1]], qb, sq).wait()

    def _scatter_drain(slot, b):
        mb, ss = bufs[b][2], bufs[b][5]
        pltpu.make_async_copy(mb, aggr_sh.at[idx4.at[slot, 1]], ss).wait()

    def _relu_sum(b):
        pb, qb, mb = bufs[b][0], bufs[b][1], bufs[b][2]

        himask = jnp.int32(-65536)

        @plsc.parallel_loop(0, K, 1, unroll=4)
        def _row(r):
            for g in range(F // 32):
                pw = pb[r, pl.ds(g * 16, 16)]
                qw = qb[r, pl.ds(g * 16, 16)]
                plo = lax.bitcast_convert_type(pw << 16, _f32)
                qlo = lax.bitcast_convert_type(qw << 16, _f32)
                phi = lax.bitcast_convert_type(pw & himask, _f32)
                qhi = lax.bitcast_convert_type(qw & himask, _f32)
                mb[r, pl.ds(g * 32, 16)] = jnp.maximum(plo + qlo, 0.0)
                mb[r, pl.ds(g * 32 + 16, 16)] = jnp.maximum(phi + qhi, 0.0)

    _idx_issue(0, 0)
    _idx_issue(1, 1)
    off = pl.multiple_of(s * OS, 8)

    @pl.when(s < NS - 1)
    def _zero():
        pltpu.sync_copy(zero_hbm.at[pl.ds(0, OS)], aggr_sh.at[pl.ds(off, OS)])

    @pl.when(s == NS - 1)
    def _zero_last():
        pltpu.sync_copy(zero_hbm.at[pl.ds(0, LAST_OS)],
                        aggr_sh.at[pl.ds(off, LAST_OS)])

    plsc.subcore_barrier()

    _idx_wait(0, 0)
    _gather_issue(0, 0)
    _idx_issue(2, 2)

    @pl.loop(0, LOOP_END, step=4)
    def _main(k):
        for b in range(4):
            b2 = b % 2
            kk = k + b
            _gather_wait(b, b2)
            if b < 2:
                @pl.when(k > 0)
                def _drain():
                    _scatter_drain((b + 2) % 4, b2)
            else:
                _scatter_drain((b + 2) % 4, b2)
            _relu_sum(b2)
            mb, ss = bufs[b2][2], bufs[b2][5]
            pltpu.async_copy(mb, aggr_sh.at[idx4.at[b, 1]], ss, add=True)
            _idx_wait(kk + 1, (b + 1) % 4)
            _gather_issue((b + 1) % 4, 1 - b2)
            _idx_issue(kk + 2, (b + 2) % 4)

    for kk in range(LOOP_END, NCHUNK):
        slot = kk % 4
        b2 = kk % 2
        _gather_wait(slot, b2)
        _scatter_drain((slot + 2) % 4, b2)
        _relu_sum(b2)
        mb, ss = bufs[b2][2], bufs[b2][5]
        pltpu.async_copy(mb, aggr_sh.at[idx4.at[slot, 1]], ss, add=True)
        if kk + 1 < NCHUNK:
            _idx_wait(kk + 1, (slot + 1) % 4)
            _gather_issue((slot + 1) % 4, 1 - b2)
        if kk + 2 < NCHUNK:
            _idx_issue(kk + 2, (slot + 2) % 4)
    for kk in (NCHUNK - 2, NCHUNK - 1):
        _scatter_drain(kk % 4, kk % 2)

    plsc.subcore_barrier()
    off = pl.multiple_of(s * OS, 8)

    @pl.when(s < NS - 1)
    def _copy_out():
        pltpu.sync_copy(aggr_sh.at[pl.ds(off, OS)],
                        out_hbm.at[c, pl.ds(off, OS)])

    @pl.when(s == NS - 1)
    def _copy_out_last():
        pltpu.sync_copy(aggr_sh.at[pl.ds(off, LAST_OS)],
                        out_hbm.at[c, pl.ds(off, LAST_OS)])


@functools.cache
def _get_sc_edge():
    return pl.kernel(
        _sc_edge_body,
        out_type=jax.ShapeDtypeStruct((NC, N, F), _f32),
        mesh=plsc.VectorSubcoreMesh(core_axis_name="c", subcore_axis_name="s",
                                    num_cores=NC, num_subcores=NS),
        compiler_params=pltpu.CompilerParams(use_tc_tiling_on_sc=False),
        scratch_types=(
            [pltpu.VMEM_SHARED((N, F), _f32)]
            + [pltpu.VMEM((4, 2, K), jnp.int32)]
            + [pltpu.VMEM((K, F // 2), jnp.int32),
               pltpu.VMEM((K, F // 2), jnp.int32),
               pltpu.VMEM((K, F), _f32)] * 2
            + [pltpu.SemaphoreType.DMA] * 10
        ),
    )


def kernel(x, edge_index, batch, W_lin2, b_lin2, We1, be1, W1a, b1a, W1b, b1b,
           W_lin3, b_lin3, We2, be2, W2a, b2a, W2b, b2b, W_lin1, b_lin1):
    eidx = (edge_index.reshape(2, NW, NCHUNK, K)
            .transpose(1, 2, 0, 3).reshape(NW, 2 * NCHUNK, K))
    zero = jnp.zeros((LAST_OS, F), _f32)
    sds = jax.ShapeDtypeStruct

    p1, q1 = pl.pallas_call(
        _tc_fold1,
        out_shape=[sds((N, F // 2), jnp.int32), sds((N, F // 2), jnp.int32)],
    )(x, W_lin2, b_lin2.reshape(1, H), We1, be1.reshape(1, F))

    sc_edge = _get_sc_edge()
    pa = sc_edge(p1, q1, eidx, zero)

    p2, q2, xr = pl.pallas_call(
        _tc_mid,
        out_shape=[sds((N, H // 2), jnp.int32), sds((N, H // 2), jnp.int32),
                   sds((N, H), _f32)],
    )(x, pa, W1a, W1a[_PERM], b1a.reshape(1, H), W1b, b1b.reshape(1, H),
      W_lin3, b_lin3.reshape(1, H), We2, be2.reshape(1, H))

    pb = sc_edge(p2, q2, eidx, zero)

    out = pl.pallas_call(
        _tc_final,
        out_shape=sds((G, C), _f32),
    )(xr, pb, W2a, W2a[_PERM], b2a.reshape(1, H), W2b, b2b.reshape(1, H),
      batch.reshape(N, 1), W_lin1, b_lin1.reshape(1, C))

    return out
